# baseline (speedup 1.0000x reference)
import functools

import jax
import jax.numpy as jnp
import numpy as np
from jax import lax
from jax.experimental import pallas as pl
from jax.experimental.pallas import tpu as pltpu
from jax.experimental.pallas import tpu_sc as plsc

N = 10000
E = 320000
F32 = jnp.float32

NC = 2
NS = 16
NW = NC * NS
EPW = E // NW
CHUNK = 80
NCHUNK = EPW // CHUNK
NP = 10240
NPT = NP // NS
CW = 64

BE = 2000
BN = 1000


def _full(shape):
    return pl.BlockSpec(shape, lambda i: tuple(0 for _ in shape))


def _make_perm():
    p = np.empty(64, np.int32)
    for k2 in range(2):
        for i in range(16):
            p[32 * k2 + 2 * i] = 32 * k2 + i
            p[32 * k2 + 2 * i + 1] = 32 * k2 + 16 + i
    return np.concatenate([p, 64 + p])


_PERM = _make_perm()
BF16 = jnp.bfloat16


def _bdiag(a, b):
    (r1, c1), (r2, c2) = a.shape, b.shape
    out = jnp.zeros((r1 + r2, c1 + c2), F32)
    out = out.at[:r1, :c1].set(a)
    out = out.at[r1:, c1:].set(b)
    return out


def _pairfold(e_ref, w1):
    e = e_ref[...]
    return jnp.maximum(jnp.dot(e, w1[...], preferred_element_type=F32), 0.0)


def _gate_out(gg):
    return jnp.reshape(gg, (gg.shape[0] * 128,)).astype(BF16)


def _gate_body(e_ref, w1, w2, g):
    t2 = _pairfold(e_ref, w1)
    g[...] = _gate_out(
        jax.nn.sigmoid(jnp.dot(t2, w2[...], preferred_element_type=F32)))


def _gate12_body(e_ref, w11, w21, w12, w2a, w2b, g1, ga, gb):
    t1 = _pairfold(e_ref, w11)
    g1[...] = _gate_out(
        jax.nn.sigmoid(jnp.dot(t1, w21[...], preferred_element_type=F32)))
    t2 = _pairfold(e_ref, w12)
    ga[...] = _gate_out(
        jax.nn.sigmoid(jnp.dot(t2, w2a[...], preferred_element_type=F32)))
    gb[...] = _gate_out(
        jax.nn.sigmoid(jnp.dot(t2, w2b[...], preferred_element_type=F32)))


_ESPEC = pl.BlockSpec((BE // 2 * 128,), lambda i: (i,))
_ESHAPE = jax.ShapeDtypeStruct((E * 64,), BF16)
_EINSPEC = pl.BlockSpec((BE // 2, 32), lambda i: (i, 0))


def _gates01(edge2, w1, w2):
    w1d = _bdiag(w1, w1)
    w2d = _bdiag(w2, w2)[:, _PERM]
    return pl.pallas_call(
        _gate_body,
        grid=(E // BE,),
        in_specs=[_EINSPEC, _full(w1d.shape), _full(w2d.shape)],
        out_specs=[_ESPEC],
        out_shape=[_ESHAPE],
    )(edge2, w1d, w2d)[0]


def _gates12(edge2, p):
    w11 = _bdiag(p["E1_1"], p["E1_1"])
    w21 = _bdiag(p["E2_1"], p["E2_1"])[:, _PERM]
    w12 = _bdiag(p["E1_2"], p["E1_2"])
    w2a = _bdiag(p["E2_2"][:, :64], p["E2_2"][:, :64])[:, _PERM]
    w2b = _bdiag(p["E2_2"][:, 64:], p["E2_2"][:, 64:])[:, _PERM]
    ws = [w11, w21, w12, w2a, w2b]
    return pl.pallas_call(
        _gate12_body,
        grid=(E // BE,),
        in_specs=[_EINSPEC] + [_full(w.shape) for w in ws],
        out_specs=[_ESPEC, _ESPEC, _ESPEC],
        out_shape=[_ESHAPE, _ESHAPE, _ESHAPE],
    )(edge2, *ws)


def _pre_body(x_ref, wm, wr, hm, hr):
    xb = x_ref[...]
    hm[...] = jnp.dot(xb, wm[...], preferred_element_type=F32)
    hr[...] = jnp.dot(xb, wr[...], preferred_element_type=F32)


def _pre(x, p):
    return pl.pallas_call(
        _pre_body,
        grid=(N // BN,),
        in_specs=[
            pl.BlockSpec((BN, 128), lambda i: (i, 0)),
            _full((128, 64)),
            _full((128, 64)),
        ],
        out_specs=[
            pl.BlockSpec((BN, 64), lambda i: (i, 0)),
            pl.BlockSpec((BN, 64), lambda i: (i, 0)),
        ],
        out_shape=[
            jax.ShapeDtypeStruct((N, 64), F32),
            jax.ShapeDtypeStruct((N, 64), F32),
        ],
    )(x, p["Wm_0"], p["Wr_0"])


NSLOT = 5
NGRP = NCHUNK // NSLOT


def _make_sc(with_deg):
    mesh = plsc.VectorSubcoreMesh(core_axis_name="c", subcore_axis_name="s")
    out_acc = jax.ShapeDtypeStruct((NC, NP, CW), F32)
    out_deg = jax.ShapeDtypeStruct((NC, NP, 16), F32)
    out_type = (out_acc, out_deg) if with_deg else out_acc
    scratch = [
        pltpu.VMEM((NSLOT, CHUNK), jnp.int32),
        pltpu.VMEM((NSLOT, CHUNK), jnp.int32),
        pltpu.VMEM((NSLOT, CHUNK, CW), F32),
        pltpu.VMEM((NSLOT, CHUNK * CW), BF16),
        pltpu.VMEM_SHARED((NP, CW), F32),
        pltpu.SemaphoreType.DMA((NSLOT,)),
        pltpu.SemaphoreType.DMA((NSLOT,)),
        pltpu.SemaphoreType.DMA((NSLOT,)),
        pltpu.SemaphoreType.DMA((NSLOT,)),
        pltpu.SemaphoreType.DMA((NSLOT,)),
    ]
    if with_deg:
        scratch += [
            pltpu.VMEM((CHUNK, 16), F32),
            pltpu.VMEM((CHUNK, 16), F32),
            pltpu.VMEM_SHARED((NP, 16), F32),
        ]

    @functools.partial(
        pl.kernel,
        out_type=out_type,
        mesh=mesh,
        scratch_types=scratch,
        compiler_params=pltpu.CompilerParams(
            use_tc_tiling_on_sc=False, needs_layout_passes=False),
    )
    def sc_fn(table, gates, src, dst, *rest):
        if with_deg:
            (out, dout, idx_s, idx_d, rows, gbuf, acc,
             sis, sid, sg, sl_, ss, ones, dbuf, dacc) = rest
        else:
            (out, idx_s, idx_d, rows, gbuf, acc,
             sis, sid, sg, sl_, ss) = rest
        c = lax.axis_index("c")
        s = lax.axis_index("s")
        wid = s * NC + c
        base = wid * EPW

        def off(i):
            return base + i * CHUNK

        def idx_s_copy(b, i):
            return pltpu.make_async_copy(
                src.at[pl.ds(off(i), CHUNK)], idx_s.at[b], sis.at[b])

        def idx_d_copy(b, i):
            return pltpu.make_async_copy(
                dst.at[pl.ds(off(i), CHUNK)], idx_d.at[b], sid.at[b])

        def gather_copy(b):
            return pltpu.make_async_copy(
                table.at[idx_s.at[b]], rows.at[b], sg.at[b])

        def gate_copy(b, i):
            return pltpu.make_async_copy(
                gates.at[pl.ds(off(i) * CW, CHUNK * CW)], gbuf.at[b], sl_.at[b])

        def scatter_copy(b):
            return pltpu.make_async_copy(
                rows.at[b], acc.at[idx_d.at[b]], ss.at[b])

        def deg_copy(b):
            return pltpu.make_async_copy(
                ones, dacc.at[idx_d.at[b]], ss.at[b])

        def _zrow(j, _):
            for k in range(CW // 16):
                rows[0, j, pl.ds(k * 16, 16)] = jnp.zeros((16,), F32)
            if with_deg:
                dbuf[j, :] = jnp.zeros((16,), F32)
                ones[j, :] = jnp.ones((16,), F32)
            return 0

        lax.fori_loop(0, CHUNK, _zrow, 0)
        for piece in range(NPT // CHUNK):
            pltpu.sync_copy(
                rows.at[0], acc.at[pl.ds(s * NPT + piece * CHUNK, CHUNK), :])
            if with_deg:
                pltpu.sync_copy(
                    dbuf, dacc.at[pl.ds(s * NPT + piece * CHUNK, CHUNK), :])
        plsc.subcore_barrier()

        for b in range(NSLOT):
            idx_s_copy(b, b).start()

        def _grp(j, _):
            i0 = j * NSLOT
            for b in range(NSLOT):
                @pl.when(j > 0)
                def _():
                    scatter_copy(b).wait()
                    if with_deg:
                        deg_copy(b).wait()
                idx_d_copy(b, i0 + b).start()
                idx_s_copy(b, i0 + b).wait()
                gather_copy(b).start()
                gate_copy(b, i0 + b).start()
            for b in range(NSLOT):
                gather_copy(b).wait()
                gate_copy(b, i0 + b).wait()

                @plsc.parallel_loop(0, CHUNK, 1, unroll=2)
                def _mul(jj):
                    for k2 in range(CW // 32):
                        q = gbuf[b, pl.ds(jj * CW + k2 * 32, 32)]
                        w = plsc.bitcast(q, jnp.int32)
                        se = plsc.bitcast(w << 16, F32)
                        so = plsc.bitcast(
                            w & jnp.int32(-65536), F32)
                        cl = k2 * 32
                        rows[b, jj, pl.ds(cl, 16)] = (
                            rows[b, jj, pl.ds(cl, 16)] * se)
                        rows[b, jj, pl.ds(cl + 16, 16)] = (
                            rows[b, jj, pl.ds(cl + 16, 16)] * so)

                idx_d_copy(b, i0 + b).wait()
                scatter_copy(b).start(add=True)
                if with_deg:
                    deg_copy(b).start(add=True)
                @pl.when(j + 1 < NGRP)
                def _():
                    idx_s_copy(b, i0 + NSLOT + b).start()
            return 0

        lax.fori_loop(0, NGRP, _grp, 0)
        for b in range(NSLOT):
            scatter_copy(b).wait()
            if with_deg:
                deg_copy(b).wait()
        plsc.subcore_barrier()
        for piece in range(NPT // CHUNK):
            rsl = pl.ds(s * NPT + piece * CHUNK, CHUNK)
            pltpu.sync_copy(acc.at[rsl, :], rows.at[piece % NSLOT])
            pltpu.sync_copy(rows.at[piece % NSLOT], out.at[c, rsl, :])
            if with_deg:
                pltpu.sync_copy(dacc.at[rsl, :], dbuf)
                pltpu.sync_copy(dbuf, dout.at[c, rsl, :])

    return sc_fn


_sc_deg = _make_sc(True)
_sc = _make_sc(False)


def _mid0_body(hr0, a0, d0, b0, wm1, wr1, hm1, hr1, inv_o):
    a = a0[...]
    d = d0[...]
    deg = d[0, :, 0:1] + d[1, :, 0:1]
    inv = 1.0 / jnp.maximum(deg, 1.0)
    agg = (a[0] + a[1]) * inv
    h1 = jnp.maximum(hr0[...] + agg + b0[...], 0.0)
    hm1[...] = jnp.dot(h1, wm1[...], preferred_element_type=F32)
    hr1[...] = jnp.dot(h1, wr1[...], preferred_element_type=F32)
    inv_o[...] = jnp.broadcast_to(inv, (inv.shape[0], 128))


def _mid0(hr0, a0, d0, p):
    return pl.pallas_call(
        _mid0_body,
        grid=(N // BN,),
        in_specs=[
            pl.BlockSpec((BN, 64), lambda i: (i, 0)),
            pl.BlockSpec((NC, BN, 64), lambda i: (0, i, 0)),
            pl.BlockSpec((NC, BN, 16), lambda i: (0, i, 0)),
            _full((1, 64)),
            _full((64, 64)),
            _full((64, 64)),
        ],
        out_specs=[
            pl.BlockSpec((BN, 64), lambda i: (i, 0)),
            pl.BlockSpec((BN, 64), lambda i: (i, 0)),
            pl.BlockSpec((BN, 128), lambda i: (i, 0)),
        ],
        out_shape=[
            jax.ShapeDtypeStruct((N, 64), F32),
            jax.ShapeDtypeStruct((N, 64), F32),
            jax.ShapeDtypeStruct((N, 128), F32),
        ],
    )(hr0, a0, d0, p["b_0"].reshape(1, -1), p["Wm_1"], p["Wr_1"])


def _mid1_body(hr1, a1, inv, b1, wm2a, wm2b, wr2, hm2a, hm2b, hr2):
    a = a1[...]
    agg = (a[0] + a[1]) * inv[:, :64]
    h2 = jnp.maximum(hr1[...] + agg + b1[...], 0.0)
    hm2a[...] = jnp.dot(h2, wm2a[...], preferred_element_type=F32)
    hm2b[...] = jnp.dot(h2, wm2b[...], preferred_element_type=F32)
    hr2[...] = jnp.dot(h2, wr2[...], preferred_element_type=F32)


def _mid1(hr1, a1, inv, p):
    return pl.pallas_call(
        _mid1_body,
        grid=(N // BN,),
        in_specs=[
            pl.BlockSpec((BN, 64), lambda i: (i, 0)),
            pl.BlockSpec((NC, BN, 64), lambda i: (0, i, 0)),
            pl.BlockSpec((BN, 128), lambda i: (i, 0)),
            _full((1, 64)),
            _full((64, 64)),
            _full((64, 64)),
            _full((64, 128)),
        ],
        out_specs=[
            pl.BlockSpec((BN, 64), lambda i: (i, 0)),
            pl.BlockSpec((BN, 64), lambda i: (i, 0)),
            pl.BlockSpec((BN, 128), lambda i: (i, 0)),
        ],
        out_shape=[
            jax.ShapeDtypeStruct((N, 64), F32),
            jax.ShapeDtypeStruct((N, 64), F32),
            jax.ShapeDtypeStruct((N, 128), F32),
        ],
    )(hr1, a1, inv, p["b_1"].reshape(1, -1),
      p["Wm_2"][:, :64], p["Wm_2"][:, 64:], p["Wr_2"])


def _final_body(hr2, a2a, a2b, inv, b2, wmu, bmu, wlv, blv,
                xw0, xb0, xw1, xb1, xw2, xb2,
                ew0, eb0, ew1, eb1, ew2, eb2,
                wox, box, woe, boe,
                outx, oute, mu_o, lv_o):
    aa = a2a[...]
    ab = a2b[...]
    agg = jnp.concatenate([aa[0] + aa[1], ab[0] + ab[1]], axis=1) * inv[...]
    h3 = hr2[...] + agg + b2[...]
    out = jnp.maximum(h3, 0.0)
    mu = jnp.clip(jnp.dot(out, wmu[...], preferred_element_type=F32) + bmu[...], -1.0, 1.0)
    lv = jnp.clip(jnp.dot(out, wlv[...], preferred_element_type=F32) + blv[...], -1.0, 1.0)

    def mlp(z, w0, b0, w1, b1, w2, b2):
        h = jnp.maximum(jnp.dot(z, w0[...], preferred_element_type=F32) + b0[...], 0.0)
        h = jnp.maximum(jnp.dot(h, w1[...], preferred_element_type=F32) + b1[...], 0.0)
        return jnp.dot(h, w2[...], preferred_element_type=F32) + b2[...]

    dx = mlp(mu, xw0, xb0, xw1, xb1, xw2, xb2)
    de = mlp(mu, ew0, eb0, ew1, eb1, ew2, eb2)
    outx[...] = jnp.dot(dx, wox[...], preferred_element_type=F32) + box[...]
    oute[...] = jnp.dot(de, woe[...], preferred_element_type=F32) + boe[...]
    mu_o[...] = mu
    lv_o[...] = lv


def _final(hr2, a2a, a2b, inv, p):
    ws = [
        p["b_2"].reshape(1, -1),
        p["Wmu"], p["bmu"].reshape(1, -1), p["Wlv"], p["blv"].reshape(1, -1),
        p["dx_W0"], p["dx_b0"].reshape(1, -1),
        p["dx_W1"], p["dx_b1"].reshape(1, -1),
        p["dx_W2"], p["dx_b2"].reshape(1, -1),
        p["de_W0"], p["de_b0"].reshape(1, -1),
        p["de_W1"], p["de_b1"].reshape(1, -1),
        p["de_W2"], p["de_b2"].reshape(1, -1),
        p["Wox"], p["box"].reshape(1, -1),
        p["Woe"], p["boe"].reshape(1, -1),
    ]
    out_x_dim = p["Wox"].shape[1]
    out_e_dim = p["Woe"].shape[1]
    aspec = pl.BlockSpec((NC, BN, 64), lambda i: (0, i, 0))
    return pl.pallas_call(
        _final_body,
        grid=(N // BN,),
        in_specs=[
            pl.BlockSpec((BN, 128), lambda i: (i, 0)),
            aspec,
            aspec,
            pl.BlockSpec((BN, 128), lambda i: (i, 0)),
        ] + [_full(w.shape) for w in ws],
        out_specs=[
            pl.BlockSpec((BN, out_x_dim), lambda i: (i, 0)),
            pl.BlockSpec((BN, out_e_dim), lambda i: (i, 0)),
            pl.BlockSpec((BN, 64), lambda i: (i, 0)),
            pl.BlockSpec((BN, 64), lambda i: (i, 0)),
        ],
        out_shape=[
            jax.ShapeDtypeStruct((N, out_x_dim), F32),
            jax.ShapeDtypeStruct((N, out_e_dim), F32),
            jax.ShapeDtypeStruct((N, 64), F32),
            jax.ShapeDtypeStruct((N, 64), F32),
        ],
    )(hr2, a2a, a2b, inv, *ws)


def kernel(x, adj, edge, params):
    p = params
    src = adj[0]
    dst = adj[1]

    edge2 = edge.reshape(E // 2, 32)
    g0 = _gates01(edge2, p["E1_0"], p["E2_0"])
    hm0, hr0 = _pre(x, p)
    a0, d0 = _sc_deg(hm0, g0, src, dst)
    g1, g2a, g2b = _gates12(edge2, p)
    hm1, hr1, inv = _mid0(hr0, a0, d0, p)
    a1 = _sc(hm1, g1, src, dst)
    hm2a, hm2b, hr2 = _mid1(hr1, a1, inv, p)
    a2a = _sc(hm2a, g2a, src, dst)
    a2b = _sc(hm2b, g2b, src, dst)
    out_x, out_e, mu, lv = _final(hr2, a2a, a2b, inv, p)
    return out_x, out_e.reshape(N, 30, 18), mu, lv


# speedup vs baseline: 1.3926x; 1.3926x over previous
import functools

import jax
import jax.numpy as jnp
from jax import lax
from jax.experimental import pallas as pl
from jax.experimental.pallas import tpu as pltpu
from jax.experimental.pallas import tpu_sc as plsc

N = 10000
E = 320000
F32 = jnp.float32

NC = 2
NS = 16
NW = NC * NS
EPW = E // NW
CHUNK = 80
NCHUNK = EPW // CHUNK
NP = 10240
NPT = NP // NS
CW = 64

BE = 2000
BN = 1000


def _full(shape):
    return pl.BlockSpec(shape, lambda i: tuple(0 for _ in shape))


def _bdiag(a, b):
    (r1, c1), (r2, c2) = a.shape, b.shape
    out = jnp.zeros((r1 + r2, c1 + c2), F32)
    out = out.at[:r1, :c1].set(a)
    out = out.at[r1:, c1:].set(b)
    return out


def _pairfold(e_ref, w1):
    e = e_ref[...]
    return jnp.maximum(jnp.dot(e, w1[...], preferred_element_type=F32), 0.0)


def _gate_out(gg):
    return jnp.reshape(gg, (gg.shape[0] * 128,))


def _gate_body(e_ref, w1, w2, g):
    t2 = _pairfold(e_ref, w1)
    g[...] = _gate_out(
        jax.nn.sigmoid(jnp.dot(t2, w2[...], preferred_element_type=F32)))


def _gate12_body(e_ref, w11, w21, w12, w2a, w2b, g1, ga, gb):
    t1 = _pairfold(e_ref, w11)
    g1[...] = _gate_out(
        jax.nn.sigmoid(jnp.dot(t1, w21[...], preferred_element_type=F32)))
    t2 = _pairfold(e_ref, w12)
    ga[...] = _gate_out(
        jax.nn.sigmoid(jnp.dot(t2, w2a[...], preferred_element_type=F32)))
    gb[...] = _gate_out(
        jax.nn.sigmoid(jnp.dot(t2, w2b[...], preferred_element_type=F32)))


_ESPEC = pl.BlockSpec((BE // 2 * 128,), lambda i: (i,))
_ESHAPE = jax.ShapeDtypeStruct((E * 64,), F32)
_EINSPEC = pl.BlockSpec((BE // 2, 32), lambda i: (i, 0))


def _gates01(edge2, w1, w2):
    w1d = _bdiag(w1, w1)
    w2d = _bdiag(w2, w2)
    return pl.pallas_call(
        _gate_body,
        grid=(E // BE,),
        in_specs=[_EINSPEC, _full(w1d.shape), _full(w2d.shape)],
        out_specs=[_ESPEC],
        out_shape=[_ESHAPE],
    )(edge2, w1d, w2d)[0]


def _gates12(edge2, p):
    w11 = _bdiag(p["E1_1"], p["E1_1"])
    w21 = _bdiag(p["E2_1"], p["E2_1"])
    w12 = _bdiag(p["E1_2"], p["E1_2"])
    w2a = _bdiag(p["E2_2"][:, :64], p["E2_2"][:, :64])
    w2b = _bdiag(p["E2_2"][:, 64:], p["E2_2"][:, 64:])
    ws = [w11, w21, w12, w2a, w2b]
    return pl.pallas_call(
        _gate12_body,
        grid=(E // BE,),
        in_specs=[_EINSPEC] + [_full(w.shape) for w in ws],
        out_specs=[_ESPEC, _ESPEC, _ESPEC],
        out_shape=[_ESHAPE, _ESHAPE, _ESHAPE],
    )(edge2, *ws)


def _pre_body(x_ref, wm, wr, hm, hr):
    xb = x_ref[...]
    hm[...] = jnp.dot(xb, wm[...], preferred_element_type=F32)
    hr[...] = jnp.dot(xb, wr[...], preferred_element_type=F32)


def _pre(x, p):
    return pl.pallas_call(
        _pre_body,
        grid=(N // BN,),
        in_specs=[
            pl.BlockSpec((BN, 128), lambda i: (i, 0)),
            _full((128, 64)),
            _full((128, 64)),
        ],
        out_specs=[
            pl.BlockSpec((BN, 64), lambda i: (i, 0)),
            pl.BlockSpec((BN, 64), lambda i: (i, 0)),
        ],
        out_shape=[
            jax.ShapeDtypeStruct((N, 64), F32),
            jax.ShapeDtypeStruct((N, 64), F32),
        ],
    )(x, p["Wm_0"], p["Wr_0"])


NSLOT = 5
NGRP = NCHUNK // NSLOT


def _make_sc(with_deg):
    mesh = plsc.VectorSubcoreMesh(core_axis_name="c", subcore_axis_name="s")
    out_acc = jax.ShapeDtypeStruct((NC, NP, CW), F32)
    out_deg = jax.ShapeDtypeStruct((NC, NP, 16), F32)
    out_type = (out_acc, out_deg) if with_deg else out_acc
    scratch = [
        pltpu.VMEM((NSLOT, CHUNK), jnp.int32),
        pltpu.VMEM((NSLOT, CHUNK), jnp.int32),
        pltpu.VMEM((NSLOT, CHUNK, CW), F32),
        pltpu.VMEM((NSLOT, CHUNK * CW), F32),
        pltpu.VMEM_SHARED((NP, CW), F32),
        pltpu.SemaphoreType.DMA((NSLOT,)),
        pltpu.SemaphoreType.DMA((NSLOT,)),
        pltpu.SemaphoreType.DMA((NSLOT,)),
        pltpu.SemaphoreType.DMA((NSLOT,)),
        pltpu.SemaphoreType.DMA((NSLOT,)),
    ]
    if with_deg:
        scratch += [
            pltpu.VMEM((CHUNK, 16), F32),
            pltpu.VMEM((CHUNK, 16), F32),
            pltpu.VMEM_SHARED((NP, 16), F32),
        ]

    @functools.partial(
        pl.kernel,
        out_type=out_type,
        mesh=mesh,
        scratch_types=scratch,
        compiler_params=pltpu.CompilerParams(use_tc_tiling_on_sc=False),
    )
    def sc_fn(table, gates, src, dst, *rest):
        if with_deg:
            (out, dout, idx_s, idx_d, rows, gbuf, acc,
             sis, sid, sg, sl_, ss, ones, dbuf, dacc) = rest
        else:
            (out, idx_s, idx_d, rows, gbuf, acc,
             sis, sid, sg, sl_, ss) = rest
        c = lax.axis_index("c")
        s = lax.axis_index("s")
        wid = s * NC + c
        base = wid * EPW

        def off(i):
            return base + i * CHUNK

        def idx_s_copy(b, i):
            return pltpu.make_async_copy(
                src.at[pl.ds(off(i), CHUNK)], idx_s.at[b], sis.at[b])

        def idx_d_copy(b, i):
            return pltpu.make_async_copy(
                dst.at[pl.ds(off(i), CHUNK)], idx_d.at[b], sid.at[b])

        def gather_copy(b):
            return pltpu.make_async_copy(
                table.at[idx_s.at[b]], rows.at[b], sg.at[b])

        def gate_copy(b, i):
            return pltpu.make_async_copy(
                gates.at[pl.ds(off(i) * CW, CHUNK * CW)], gbuf.at[b], sl_.at[b])

        def scatter_copy(b):
            return pltpu.make_async_copy(
                rows.at[b], acc.at[idx_d.at[b]], ss.at[b])

        def deg_copy(b):
            return pltpu.make_async_copy(
                ones, dacc.at[idx_d.at[b]], ss.at[b])

        def _zrow(j, _):
            for k in range(CW // 16):
                rows[0, j, pl.ds(k * 16, 16)] = jnp.zeros((16,), F32)
            if with_deg:
                dbuf[j, :] = jnp.zeros((16,), F32)
                ones[j, :] = jnp.ones((16,), F32)
            return 0

        lax.fori_loop(0, CHUNK, _zrow, 0)
        for piece in range(NPT // CHUNK):
            pltpu.sync_copy(
                rows.at[0], acc.at[pl.ds(s * NPT + piece * CHUNK, CHUNK), :])
            if with_deg:
                pltpu.sync_copy(
                    dbuf, dacc.at[pl.ds(s * NPT + piece * CHUNK, CHUNK), :])
        plsc.subcore_barrier()

        for b in range(NSLOT):
            idx_s_copy(b, b).start()

        def _grp(j, _):
            i0 = j * NSLOT
            for b in range(NSLOT):
                @pl.when(j > 0)
                def _():
                    scatter_copy(b).wait()
                    if with_deg:
                        deg_copy(b).wait()
                idx_d_copy(b, i0 + b).start()
                idx_s_copy(b, i0 + b).wait()
                gather_copy(b).start()
                gate_copy(b, i0 + b).start()
            for b in range(NSLOT):
                gather_copy(b).wait()
                gate_copy(b, i0 + b).wait()

                @plsc.parallel_loop(0, CHUNK, 1, unroll=2)
                def _mul(jj):
                    for k in range(CW // 16):
                        sl = pl.ds(k * 16, 16)
                        rows[b, jj, sl] = (
                            rows[b, jj, sl]
                            * gbuf[b, pl.ds(jj * CW + k * 16, 16)])

                idx_d_copy(b, i0 + b).wait()
                scatter_copy(b).start(add=True)
                if with_deg:
                    deg_copy(b).start(add=True)
                @pl.when(j + 1 < NGRP)
                def _():
                    idx_s_copy(b, i0 + NSLOT + b).start()
            return 0

        lax.fori_loop(0, NGRP, _grp, 0)
        for b in range(NSLOT):
            scatter_copy(b).wait()
            if with_deg:
                deg_copy(b).wait()
        plsc.subcore_barrier()
        for piece in range(NPT // CHUNK):
            rsl = pl.ds(s * NPT + piece * CHUNK, CHUNK)
            pltpu.sync_copy(acc.at[rsl, :], rows.at[piece % NSLOT])
            pltpu.sync_copy(rows.at[piece % NSLOT], out.at[c, rsl, :])
            if with_deg:
                pltpu.sync_copy(dacc.at[rsl, :], dbuf)
                pltpu.sync_copy(dbuf, dout.at[c, rsl, :])

    return sc_fn


_sc_deg = _make_sc(True)
_sc = _make_sc(False)


def _mid0_body(hr0, a0, d0, b0, wm1, wr1, hm1, hr1, inv_o):
    a = a0[...]
    d = d0[...]
    deg = d[0, :, 0:1] + d[1, :, 0:1]
    inv = 1.0 / jnp.maximum(deg, 1.0)
    agg = (a[0] + a[1]) * inv
    h1 = jnp.maximum(hr0[...] + agg + b0[...], 0.0)
    hm1[...] = jnp.dot(h1, wm1[...], preferred_element_type=F32)
    hr1[...] = jnp.dot(h1, wr1[...], preferred_element_type=F32)
    inv_o[...] = jnp.broadcast_to(inv, (inv.shape[0], 128))


def _mid0(hr0, a0, d0, p):
    return pl.pallas_call(
        _mid0_body,
        grid=(N // BN,),
        in_specs=[
            pl.BlockSpec((BN, 64), lambda i: (i, 0)),
            pl.BlockSpec((NC, BN, 64), lambda i: (0, i, 0)),
            pl.BlockSpec((NC, BN, 16), lambda i: (0, i, 0)),
            _full((1, 64)),
            _full((64, 64)),
            _full((64, 64)),
        ],
        out_specs=[
            pl.BlockSpec((BN, 64), lambda i: (i, 0)),
            pl.BlockSpec((BN, 64), lambda i: (i, 0)),
            pl.BlockSpec((BN, 128), lambda i: (i, 0)),
        ],
        out_shape=[
            jax.ShapeDtypeStruct((N, 64), F32),
            jax.ShapeDtypeStruct((N, 64), F32),
            jax.ShapeDtypeStruct((N, 128), F32),
        ],
    )(hr0, a0, d0, p["b_0"].reshape(1, -1), p["Wm_1"], p["Wr_1"])


def _mid1_body(hr1, a1, inv, b1, wm2a, wm2b, wr2, hm2a, hm2b, hr2):
    a = a1[...]
    agg = (a[0] + a[1]) * inv[:, :64]
    h2 = jnp.maximum(hr1[...] + agg + b1[...], 0.0)
    hm2a[...] = jnp.dot(h2, wm2a[...], preferred_element_type=F32)
    hm2b[...] = jnp.dot(h2, wm2b[...], preferred_element_type=F32)
    hr2[...] = jnp.dot(h2, wr2[...], preferred_element_type=F32)


def _mid1(hr1, a1, inv, p):
    return pl.pallas_call(
        _mid1_body,
        grid=(N // BN,),
        in_specs=[
            pl.BlockSpec((BN, 64), lambda i: (i, 0)),
            pl.BlockSpec((NC, BN, 64), lambda i: (0, i, 0)),
            pl.BlockSpec((BN, 128), lambda i: (i, 0)),
            _full((1, 64)),
            _full((64, 64)),
            _full((64, 64)),
            _full((64, 128)),
        ],
        out_specs=[
            pl.BlockSpec((BN, 64), lambda i: (i, 0)),
            pl.BlockSpec((BN, 64), lambda i: (i, 0)),
            pl.BlockSpec((BN, 128), lambda i: (i, 0)),
        ],
        out_shape=[
            jax.ShapeDtypeStruct((N, 64), F32),
            jax.ShapeDtypeStruct((N, 64), F32),
            jax.ShapeDtypeStruct((N, 128), F32),
        ],
    )(hr1, a1, inv, p["b_1"].reshape(1, -1),
      p["Wm_2"][:, :64], p["Wm_2"][:, 64:], p["Wr_2"])


def _final_body(hr2, a2a, a2b, inv, b2, wmu, bmu, wlv, blv,
                xw0, xb0, xw1, xb1, xw2, xb2,
                ew0, eb0, ew1, eb1, ew2, eb2,
                wox, box, woe, boe,
                outx, oute, mu_o, lv_o):
    aa = a2a[...]
    ab = a2b[...]
    agg = jnp.concatenate([aa[0] + aa[1], ab[0] + ab[1]], axis=1) * inv[...]
    h3 = hr2[...] + agg + b2[...]
    out = jnp.maximum(h3, 0.0)
    mu = jnp.clip(jnp.dot(out, wmu[...], preferred_element_type=F32) + bmu[...], -1.0, 1.0)
    lv = jnp.clip(jnp.dot(out, wlv[...], preferred_element_type=F32) + blv[...], -1.0, 1.0)

    def mlp(z, w0, b0, w1, b1, w2, b2):
        h = jnp.maximum(jnp.dot(z, w0[...], preferred_element_type=F32) + b0[...], 0.0)
        h = jnp.maximum(jnp.dot(h, w1[...], preferred_element_type=F32) + b1[...], 0.0)
        return jnp.dot(h, w2[...], preferred_element_type=F32) + b2[...]

    dx = mlp(mu, xw0, xb0, xw1, xb1, xw2, xb2)
    de = mlp(mu, ew0, eb0, ew1, eb1, ew2, eb2)
    outx[...] = jnp.dot(dx, wox[...], preferred_element_type=F32) + box[...]
    oute[...] = jnp.dot(de, woe[...], preferred_element_type=F32) + boe[...]
    mu_o[...] = mu
    lv_o[...] = lv


def _final(hr2, a2a, a2b, inv, p):
    ws = [
        p["b_2"].reshape(1, -1),
        p["Wmu"], p["bmu"].reshape(1, -1), p["Wlv"], p["blv"].reshape(1, -1),
        p["dx_W0"], p["dx_b0"].reshape(1, -1),
        p["dx_W1"], p["dx_b1"].reshape(1, -1),
        p["dx_W2"], p["dx_b2"].reshape(1, -1),
        p["de_W0"], p["de_b0"].reshape(1, -1),
        p["de_W1"], p["de_b1"].reshape(1, -1),
        p["de_W2"], p["de_b2"].reshape(1, -1),
        p["Wox"], p["box"].reshape(1, -1),
        p["Woe"], p["boe"].reshape(1, -1),
    ]
    out_x_dim = p["Wox"].shape[1]
    out_e_dim = p["Woe"].shape[1]
    aspec = pl.BlockSpec((NC, BN, 64), lambda i: (0, i, 0))
    return pl.pallas_call(
        _final_body,
        grid=(N // BN,),
        in_specs=[
            pl.BlockSpec((BN, 128), lambda i: (i, 0)),
            aspec,
            aspec,
            pl.BlockSpec((BN, 128), lambda i: (i, 0)),
        ] + [_full(w.shape) for w in ws],
        out_specs=[
            pl.BlockSpec((BN, out_x_dim), lambda i: (i, 0)),
            pl.BlockSpec((BN, out_e_dim), lambda i: (i, 0)),
            pl.BlockSpec((BN, 64), lambda i: (i, 0)),
            pl.BlockSpec((BN, 64), lambda i: (i, 0)),
        ],
        out_shape=[
            jax.ShapeDtypeStruct((N, out_x_dim), F32),
            jax.ShapeDtypeStruct((N, out_e_dim), F32),
            jax.ShapeDtypeStruct((N, 64), F32),
            jax.ShapeDtypeStruct((N, 64), F32),
        ],
    )(hr2, a2a, a2b, inv, *ws)


def kernel(x, adj, edge, params):
    p = params
    src = adj[0]
    dst = adj[1]

    edge2 = edge.reshape(E // 2, 32)
    g0 = _gates01(edge2, p["E1_0"], p["E2_0"])
    hm0, hr0 = _pre(x, p)
    a0, d0 = _sc_deg(hm0, g0, src, dst)
    g1, g2a, g2b = _gates12(edge2, p)
    hm1, hr1, inv = _mid0(hr0, a0, d0, p)
    a1 = _sc(hm1, g1, src, dst)
    hm2a, hm2b, hr2 = _mid1(hr1, a1, inv, p)
    a2a = _sc(hm2a, g2a, src, dst)
    a2b = _sc(hm2b, g2b, src, dst)
    out_x, out_e, mu, lv = _final(hr2, a2a, a2b, inv, p)
    return out_x, out_e.reshape(N, 30, 18), mu, lv


# speedup vs baseline: 1.5400x; 1.1059x over previous
import functools

import jax
import jax.numpy as jnp
from jax import lax
from jax.experimental import pallas as pl
from jax.experimental.pallas import tpu as pltpu
from jax.experimental.pallas import tpu_sc as plsc

N = 10000
E = 320000
F32 = jnp.float32

NC = 2
NS = 16
NW = NC * NS
EPW = E // NW
CHUNK = 80
NCHUNK = EPW // CHUNK
NP = 10240
NPT = NP // NS
CW = 64

BE = 3200
BN = 1000


def _full(shape):
    return pl.BlockSpec(shape, lambda i: tuple(0 for _ in shape))


def _bdiag8(m):
    r, c = m.shape
    out = jnp.zeros((8 * r, 8 * c), F32)
    for k in range(8):
        out = out.at[k * r:(k + 1) * r, k * c:(k + 1) * c].set(m)
    return out


def _octfold(e_ref, w1):
    e = e_ref[...]
    return jnp.maximum(jnp.dot(e, w1[...], preferred_element_type=F32), 0.0)


def _gate_out(gg):
    return jnp.reshape(gg, (gg.shape[0] * 512,))


def _gate_body(e_ref, w1, w2, g):
    t = _octfold(e_ref, w1)
    g[...] = _gate_out(
        jax.nn.sigmoid(jnp.dot(t, w2[...], preferred_element_type=F32)))


def _gate12_body(e_ref, w11, w21, w12, w2a, w2b, g1, ga, gb):
    t1 = _octfold(e_ref, w11)
    g1[...] = _gate_out(
        jax.nn.sigmoid(jnp.dot(t1, w21[...], preferred_element_type=F32)))
    t2 = _octfold(e_ref, w12)
    ga[...] = _gate_out(
        jax.nn.sigmoid(jnp.dot(t2, w2a[...], preferred_element_type=F32)))
    gb[...] = _gate_out(
        jax.nn.sigmoid(jnp.dot(t2, w2b[...], preferred_element_type=F32)))


_ESPEC = pl.BlockSpec((BE * 64,), lambda i: (i,))
_ESHAPE = jax.ShapeDtypeStruct((E * 64,), F32)
_EINSPEC = pl.BlockSpec((BE // 8, 128), lambda i: (i, 0))


def _gates01(edge8, w1, w2):
    w1d = _bdiag8(w1)
    w2d = _bdiag8(w2)
    return pl.pallas_call(
        _gate_body,
        grid=(E // BE,),
        in_specs=[_EINSPEC, _full(w1d.shape), _full(w2d.shape)],
        out_specs=[_ESPEC],
        out_shape=[_ESHAPE],
    )(edge8, w1d, w2d)[0]


def _gates12(edge8, p):
    w11 = _bdiag8(p["E1_1"])
    w21 = _bdiag8(p["E2_1"])
    w12 = _bdiag8(p["E1_2"])
    w2a = _bdiag8(p["E2_2"][:, :64])
    w2b = _bdiag8(p["E2_2"][:, 64:])
    ws = [w11, w21, w12, w2a, w2b]
    return pl.pallas_call(
        _gate12_body,
        grid=(E // BE,),
        in_specs=[_EINSPEC] + [_full(w.shape) for w in ws],
        out_specs=[_ESPEC, _ESPEC, _ESPEC],
        out_shape=[_ESHAPE, _ESHAPE, _ESHAPE],
    )(edge8, *ws)


def _pre_body(x_ref, wm, wr, hm, hr):
    xb = x_ref[...]
    hm[...] = jnp.dot(xb, wm[...], preferred_element_type=F32)
    hr[...] = jnp.dot(xb, wr[...], preferred_element_type=F32)


def _pre(x, p):
    return pl.pallas_call(
        _pre_body,
        grid=(N // BN,),
        in_specs=[
            pl.BlockSpec((BN, 128), lambda i: (i, 0)),
            _full((128, 64)),
            _full((128, 64)),
        ],
        out_specs=[
            pl.BlockSpec((BN, 64), lambda i: (i, 0)),
            pl.BlockSpec((BN, 64), lambda i: (i, 0)),
        ],
        out_shape=[
            jax.ShapeDtypeStruct((N, 64), F32),
            jax.ShapeDtypeStruct((N, 64), F32),
        ],
    )(x, p["Wm_0"], p["Wr_0"])


NSLOT = 5
NGRP = NCHUNK // NSLOT


def _make_sc(with_deg):
    mesh = plsc.VectorSubcoreMesh(core_axis_name="c", subcore_axis_name="s")
    out_acc = jax.ShapeDtypeStruct((NC, NP, CW), F32)
    out_deg = jax.ShapeDtypeStruct((NC, NP, 16), F32)
    out_type = (out_acc, out_deg) if with_deg else out_acc
    scratch = [
        pltpu.VMEM((NSLOT, CHUNK), jnp.int32),
        pltpu.VMEM((NSLOT, CHUNK), jnp.int32),
        pltpu.VMEM((NSLOT, CHUNK, CW), F32),
        pltpu.VMEM((NSLOT, CHUNK * CW), F32),
        pltpu.VMEM_SHARED((NP, CW), F32),
        pltpu.SemaphoreType.DMA((NSLOT,)),
        pltpu.SemaphoreType.DMA((NSLOT,)),
        pltpu.SemaphoreType.DMA((NSLOT,)),
        pltpu.SemaphoreType.DMA((NSLOT,)),
        pltpu.SemaphoreType.DMA((NSLOT,)),
    ]
    if with_deg:
        scratch += [
            pltpu.VMEM((CHUNK, 16), F32),
            pltpu.VMEM((CHUNK, 16), F32),
            pltpu.VMEM_SHARED((NP, 16), F32),
        ]

    @functools.partial(
        pl.kernel,
        out_type=out_type,
        mesh=mesh,
        scratch_types=scratch,
        compiler_params=pltpu.CompilerParams(use_tc_tiling_on_sc=False),
    )
    def sc_fn(table, gates, src, dst, *rest):
        if with_deg:
            (out, dout, idx_s, idx_d, rows, gbuf, acc,
             sis, sid, sg, sl_, ss, ones, dbuf, dacc) = rest
        else:
            (out, idx_s, idx_d, rows, gbuf, acc,
             sis, sid, sg, sl_, ss) = rest
        c = lax.axis_index("c")
        s = lax.axis_index("s")
        wid = s * NC + c
        base = wid * EPW

        def off(i):
            return base + i * CHUNK

        def idx_s_copy(b, i):
            return pltpu.make_async_copy(
                src.at[pl.ds(off(i), CHUNK)], idx_s.at[b], sis.at[b])

        def idx_d_copy(b, i):
            return pltpu.make_async_copy(
                dst.at[pl.ds(off(i), CHUNK)], idx_d.at[b], sid.at[b])

        def gather_copy(b):
            return pltpu.make_async_copy(
                table.at[idx_s.at[b]], rows.at[b], sg.at[b])

        def gate_copy(b, i):
            return pltpu.make_async_copy(
                gates.at[pl.ds(off(i) * CW, CHUNK * CW)], gbuf.at[b], sl_.at[b])

        def scatter_copy(b):
            return pltpu.make_async_copy(
                rows.at[b], acc.at[idx_d.at[b]], ss.at[b])

        def deg_copy(b):
            return pltpu.make_async_copy(
                ones, dacc.at[idx_d.at[b]], ss.at[b])

        def _zrow(j, _):
            for k in range(CW // 16):
                rows[0, j, pl.ds(k * 16, 16)] = jnp.zeros((16,), F32)
            if with_deg:
                dbuf[j, :] = jnp.zeros((16,), F32)
                ones[j, :] = jnp.ones((16,), F32)
            return 0

        lax.fori_loop(0, CHUNK, _zrow, 0)
        for piece in range(NPT // CHUNK):
            pltpu.sync_copy(
                rows.at[0], acc.at[pl.ds(s * NPT + piece * CHUNK, CHUNK), :])
            if with_deg:
                pltpu.sync_copy(
                    dbuf, dacc.at[pl.ds(s * NPT + piece * CHUNK, CHUNK), :])
        plsc.subcore_barrier()

        for b in range(NSLOT):
            idx_s_copy(b, b).start()

        def _grp(j, _):
            i0 = j * NSLOT
            for b in range(NSLOT):
                @pl.when(j > 0)
                def _():
                    scatter_copy(b).wait()
                    if with_deg:
                        deg_copy(b).wait()
                idx_d_copy(b, i0 + b).start()
                idx_s_copy(b, i0 + b).wait()
                gather_copy(b).start()
                gate_copy(b, i0 + b).start()
            for b in range(NSLOT):
                gather_copy(b).wait()
                gate_copy(b, i0 + b).wait()

                @plsc.parallel_loop(0, CHUNK, 1, unroll=2)
                def _mul(jj):
                    for k in range(CW // 16):
                        sl = pl.ds(k * 16, 16)
                        rows[b, jj, sl] = (
                            rows[b, jj, sl]
                            * gbuf[b, pl.ds(jj * CW + k * 16, 16)])

                idx_d_copy(b, i0 + b).wait()
                scatter_copy(b).start(add=True)
                if with_deg:
                    deg_copy(b).start(add=True)
                @pl.when(j + 1 < NGRP)
                def _():
                    idx_s_copy(b, i0 + NSLOT + b).start()
            return 0

        lax.fori_loop(0, NGRP, _grp, 0)
        for b in range(NSLOT):
            scatter_copy(b).wait()
            if with_deg:
                deg_copy(b).wait()
        plsc.subcore_barrier()
        for piece in range(NPT // CHUNK):
            rsl = pl.ds(s * NPT + piece * CHUNK, CHUNK)
            pltpu.sync_copy(acc.at[rsl, :], rows.at[piece % NSLOT])
            pltpu.sync_copy(rows.at[piece % NSLOT], out.at[c, rsl, :])
            if with_deg:
                pltpu.sync_copy(dacc.at[rsl, :], dbuf)
                pltpu.sync_copy(dbuf, dout.at[c, rsl, :])

    return sc_fn


_sc_deg = _make_sc(True)
_sc = _make_sc(False)


def _mid0_body(hr0, a0, d0, b0, wm1, wr1, hm1, hr1, inv_o):
    a = a0[...]
    d = d0[...]
    deg = d[0, :, 0:1] + d[1, :, 0:1]
    inv = 1.0 / jnp.maximum(deg, 1.0)
    agg = (a[0] + a[1]) * inv
    h1 = jnp.maximum(hr0[...] + agg + b0[...], 0.0)
    hm1[...] = jnp.dot(h1, wm1[...], preferred_element_type=F32)
    hr1[...] = jnp.dot(h1, wr1[...], preferred_element_type=F32)
    inv_o[...] = jnp.broadcast_to(inv, (inv.shape[0], 128))


def _mid0(hr0, a0, d0, p):
    return pl.pallas_call(
        _mid0_body,
        grid=(N // BN,),
        in_specs=[
            pl.BlockSpec((BN, 64), lambda i: (i, 0)),
            pl.BlockSpec((NC, BN, 64), lambda i: (0, i, 0)),
            pl.BlockSpec((NC, BN, 16), lambda i: (0, i, 0)),
            _full((1, 64)),
            _full((64, 64)),
            _full((64, 64)),
        ],
        out_specs=[
            pl.BlockSpec((BN, 64), lambda i: (i, 0)),
            pl.BlockSpec((BN, 64), lambda i: (i, 0)),
            pl.BlockSpec((BN, 128), lambda i: (i, 0)),
        ],
        out_shape=[
            jax.ShapeDtypeStruct((N, 64), F32),
            jax.ShapeDtypeStruct((N, 64), F32),
            jax.ShapeDtypeStruct((N, 128), F32),
        ],
    )(hr0, a0, d0, p["b_0"].reshape(1, -1), p["Wm_1"], p["Wr_1"])


def _mid1_body(hr1, a1, inv, b1, wm2a, wm2b, wr2, hm2a, hm2b, hr2):
    a = a1[...]
    agg = (a[0] + a[1]) * inv[:, :64]
    h2 = jnp.maximum(hr1[...] + agg + b1[...], 0.0)
    hm2a[...] = jnp.dot(h2, wm2a[...], preferred_element_type=F32)
    hm2b[...] = jnp.dot(h2, wm2b[...], preferred_element_type=F32)
    hr2[...] = jnp.dot(h2, wr2[...], preferred_element_type=F32)


def _mid1(hr1, a1, inv, p):
    return pl.pallas_call(
        _mid1_body,
        grid=(N // BN,),
        in_specs=[
            pl.BlockSpec((BN, 64), lambda i: (i, 0)),
            pl.BlockSpec((NC, BN, 64), lambda i: (0, i, 0)),
            pl.BlockSpec((BN, 128), lambda i: (i, 0)),
            _full((1, 64)),
            _full((64, 64)),
            _full((64, 64)),
            _full((64, 128)),
        ],
        out_specs=[
            pl.BlockSpec((BN, 64), lambda i: (i, 0)),
            pl.BlockSpec((BN, 64), lambda i: (i, 0)),
            pl.BlockSpec((BN, 128), lambda i: (i, 0)),
        ],
        out_shape=[
            jax.ShapeDtypeStruct((N, 64), F32),
            jax.ShapeDtypeStruct((N, 64), F32),
            jax.ShapeDtypeStruct((N, 128), F32),
        ],
    )(hr1, a1, inv, p["b_1"].reshape(1, -1),
      p["Wm_2"][:, :64], p["Wm_2"][:, 64:], p["Wr_2"])


def _final_body(hr2, a2a, a2b, inv, b2, wmu, bmu, wlv, blv,
                xw0, xb0, xw1, xb1, xw2, xb2,
                ew0, eb0, ew1, eb1, ew2, eb2,
                wox, box, woe, boe,
                outx, oute, mu_o, lv_o):
    aa = a2a[...]
    ab = a2b[...]
    agg = jnp.concatenate([aa[0] + aa[1], ab[0] + ab[1]], axis=1) * inv[...]
    h3 = hr2[...] + agg + b2[...]
    out = jnp.maximum(h3, 0.0)
    mu = jnp.clip(jnp.dot(out, wmu[...], preferred_element_type=F32) + bmu[...], -1.0, 1.0)
    lv = jnp.clip(jnp.dot(out, wlv[...], preferred_element_type=F32) + blv[...], -1.0, 1.0)

    def mlp(z, w0, b0, w1, b1, w2, b2):
        h = jnp.maximum(jnp.dot(z, w0[...], preferred_element_type=F32) + b0[...], 0.0)
        h = jnp.maximum(jnp.dot(h, w1[...], preferred_element_type=F32) + b1[...], 0.0)
        return jnp.dot(h, w2[...], preferred_element_type=F32) + b2[...]

    dx = mlp(mu, xw0, xb0, xw1, xb1, xw2, xb2)
    de = mlp(mu, ew0, eb0, ew1, eb1, ew2, eb2)
    outx[...] = jnp.dot(dx, wox[...], preferred_element_type=F32) + box[...]
    oute[...] = jnp.dot(de, woe[...], preferred_element_type=F32) + boe[...]
    mu_o[...] = mu
    lv_o[...] = lv


def _final(hr2, a2a, a2b, inv, p):
    ws = [
        p["b_2"].reshape(1, -1),
        p["Wmu"], p["bmu"].reshape(1, -1), p["Wlv"], p["blv"].reshape(1, -1),
        p["dx_W0"], p["dx_b0"].reshape(1, -1),
        p["dx_W1"], p["dx_b1"].reshape(1, -1),
        p["dx_W2"], p["dx_b2"].reshape(1, -1),
        p["de_W0"], p["de_b0"].reshape(1, -1),
        p["de_W1"], p["de_b1"].reshape(1, -1),
        p["de_W2"], p["de_b2"].reshape(1, -1),
        p["Wox"], p["box"].reshape(1, -1),
        p["Woe"], p["boe"].reshape(1, -1),
    ]
    out_x_dim = p["Wox"].shape[1]
    out_e_dim = p["Woe"].shape[1]
    aspec = pl.BlockSpec((NC, BN, 64), lambda i: (0, i, 0))
    return pl.pallas_call(
        _final_body,
        grid=(N // BN,),
        in_specs=[
            pl.BlockSpec((BN, 128), lambda i: (i, 0)),
            aspec,
            aspec,
            pl.BlockSpec((BN, 128), lambda i: (i, 0)),
        ] + [_full(w.shape) for w in ws],
        out_specs=[
            pl.BlockSpec((BN, out_x_dim), lambda i: (i, 0)),
            pl.BlockSpec((BN, out_e_dim), lambda i: (i, 0)),
            pl.BlockSpec((BN, 64), lambda i: (i, 0)),
            pl.BlockSpec((BN, 64), lambda i: (i, 0)),
        ],
        out_shape=[
            jax.ShapeDtypeStruct((N, out_x_dim), F32),
            jax.ShapeDtypeStruct((N, out_e_dim), F32),
            jax.ShapeDtypeStruct((N, 64), F32),
            jax.ShapeDtypeStruct((N, 64), F32),
        ],
    )(hr2, a2a, a2b, inv, *ws)


def kernel(x, adj, edge, params):
    p = params
    src = adj[0]
    dst = adj[1]

    edge8 = edge.reshape(E // 8, 128)
    g0 = _gates01(edge8, p["E1_0"], p["E2_0"])
    hm0, hr0 = _pre(x, p)
    a0, d0 = _sc_deg(hm0, g0, src, dst)
    g1, g2a, g2b = _gates12(edge8, p)
    hm1, hr1, inv = _mid0(hr0, a0, d0, p)
    a1 = _sc(hm1, g1, src, dst)
    hm2a, hm2b, hr2 = _mid1(hr1, a1, inv, p)
    a2a = _sc(hm2a, g2a, src, dst)
    a2b = _sc(hm2b, g2b, src, dst)
    out_x, out_e, mu, lv = _final(hr2, a2a, a2b, inv, p)
    return out_x, out_e.reshape(N, 30, 18), mu, lv
